# Initial kernel scaffold; baseline (speedup 1.0000x reference)
import jax
import jax.numpy as jnp
from jax.experimental import pallas as pl


def kernel(xi, memory, visible_memory, read_weights, write_weights, read_vectors, usage, least_used_mem, read_positions, W_rq, b_rq, W_wv, b_wv, W_ig, b_ig, W_wg, b_wg):
    raise NotImplementedError("write your pallas kernel here")


# speedup vs baseline: 7.5795x; 7.5795x over previous
import jax
import jax.numpy as jnp
from jax.experimental import pallas as pl
from jax.experimental.pallas import tpu as pltpu

_B, _INPUT, _MEM, _W, _R, _K = 16, 512, 100000, 32, 4, 8
_C = _R * _K + 1
_DELTA = 0.005
_EPS = 1e-6
_BM = 10000
_NM = _MEM // _BM
_BIGF = 3e38
_BIGI = 2**30


def _knn_kernel(mem_ref, q_ref, visnew_ref, rp_ref, out_ref, dist_scratch):
    i = pl.program_id(1)
    mem = mem_ref[0]
    q = q_ref[0]
    ones = jnp.ones((1, _W), dtype=jnp.float32)
    dn = (((1,), (1,)), ((), ()))
    qn = jnp.sum(q * q, axis=1, keepdims=True)
    dots = jax.lax.dot_general(q, mem, dimension_numbers=dn,
                               preferred_element_type=jnp.float32)
    norms = jax.lax.dot_general(ones, mem * mem, dimension_numbers=dn,
                                preferred_element_type=jnp.float32,
                                precision=jax.lax.Precision.HIGHEST)
    d = norms - 2.0 * dots + qn

    visn = visnew_ref[0]
    fdots = jax.lax.dot_general(q, visn, dimension_numbers=dn,
                                preferred_element_type=jnp.float32)
    fnorm = jax.lax.dot_general(ones, visn * visn, dimension_numbers=dn,
                                preferred_element_type=jnp.float32,
                                precision=jax.lax.Precision.HIGHEST)
    fvals = fnorm - 2.0 * fdots + qn

    g = jax.lax.broadcasted_iota(jnp.int32, (1, _BM), 1) + i * _BM
    for j in range(_C):
        p = rp_ref[0, 0, j]
        d = jnp.where(g == p, fvals[:, j:j + 1], d)

    dist_scratch[pl.ds(i, 1)] = d[None]

    @pl.when(i == _NM - 1)
    def _():
        D = dist_scratch[...]
        gi = (jax.lax.broadcasted_iota(jnp.int32, (_NM, 1, _BM), 0) * _BM
              + jax.lax.broadcasted_iota(jnp.int32, (_NM, 1, _BM), 2))
        Dv = D
        for k in range(_K):
            t = jnp.min(Dv, axis=2, keepdims=True)
            mv = jnp.min(t, axis=0, keepdims=True)
            cand = jnp.where(Dv == mv, jnp.broadcast_to(gi, Dv.shape), _BIGI)
            c1 = jnp.min(cand, axis=2, keepdims=True)
            idx = jnp.min(c1, axis=0)
            out_ref[0, :, k:k + 1] = idx
            Dv = jnp.where(gi == idx[None], _BIGF, Dv)


def _topk_positions(memory, read_query, visnew, read_positions):
    return pl.pallas_call(
        _knn_kernel,
        grid=(_B, _NM),
        in_specs=[
            pl.BlockSpec((1, _BM, _W), lambda b, i: (b, i, 0)),
            pl.BlockSpec((1, _R, _W), lambda b, i: (b, 0, 0)),
            pl.BlockSpec((1, _C, _W), lambda b, i: (b, 0, 0)),
            pl.BlockSpec((1, 1, _C), lambda b, i: (b, 0, 0)),
        ],
        out_specs=pl.BlockSpec((1, _R, _K), lambda b, i: (b, 0, 0)),
        out_shape=jax.ShapeDtypeStruct((_B, _R, _K), jnp.int32),
        scratch_shapes=[pltpu.VMEM((_NM, _R, _BM), jnp.float32)],
    )(memory, read_query, visnew, read_positions[:, None, :])


def kernel(xi, memory, visible_memory, read_weights, write_weights, read_vectors,
           usage, least_used_mem, read_positions, W_rq, b_rq, W_wv, b_wv,
           W_ig, b_ig, W_wg, b_wg):
    b, m, w = memory.shape
    r, c = _R, _C
    read_query = (xi @ W_rq.T + b_rq).reshape(b, r, w)
    write_vector = (xi @ W_wv.T + b_wv).reshape(b, 1, w)
    interpolation_gate = jax.nn.sigmoid(xi @ W_ig.T + b_ig).reshape(b, c)
    write_gate = jax.nn.sigmoid(xi @ W_wg.T + b_wg).reshape(b, 1)
    rw_g = jnp.take_along_axis(read_weights, read_positions, axis=1)
    rel_usage = jnp.take_along_axis(usage, read_positions, axis=1)
    minusage = jnp.min(rel_usage, axis=-1, keepdims=True)
    Imask = (rel_usage == minusage).astype(jnp.float32)
    x = interpolation_gate * rw_g
    y = (1.0 - interpolation_gate) * Imask
    ww_new = write_gate * (x + y)
    visnew = visible_memory * (1.0 - Imask[:, :, None]) + ww_new[:, :, None] * write_vector
    mem_limit_reached = least_used_mem[0, 0] >= m - 1
    least_used_mem = jnp.where(mem_limit_reached,
                               jnp.full_like(least_used_mem, c + 1),
                               least_used_mem + 1)
    positions = _topk_positions(memory, read_query, visnew, read_positions)
    new_pos = jnp.concatenate([positions.reshape(b, r * _K), least_used_mem], axis=1)
    idx3 = jnp.broadcast_to(new_pos[:, :, None], (b, c, w))
    vis = jnp.take_along_axis(memory, idx3, axis=1)
    match = read_positions[:, None, :] == new_pos[:, :, None]
    jar = jnp.arange(c, dtype=jnp.int32)
    lastj = jnp.max(jnp.where(match, jar[None, None, :], -1), axis=-1)
    vis_fresh = jnp.take_along_axis(
        visnew, jnp.broadcast_to(jnp.clip(lastj, 0, c - 1)[:, :, None], (b, c, w)), axis=1)
    vis = jnp.where((lastj >= 0)[:, :, None], vis_fresh, vis)
    dot = jnp.einsum('bcw,brw->brc', vis, read_query)
    a_norm = jnp.linalg.norm(vis, axis=2)
    q_norm = jnp.linalg.norm(read_query, axis=2)
    cos = dot / (q_norm[:, :, None] * a_norm[:, None, :] + _EPS)
    rw_soft = jax.nn.softmax(cos, axis=2)
    new_read_vectors = jnp.einsum('brc,bcw->brw', rw_soft, vis)
    return new_read_vectors
